# Initial kernel scaffold; baseline (speedup 1.0000x reference)
import jax
import jax.numpy as jnp
from jax.experimental import pallas as pl


def kernel(x, target):
    raise NotImplementedError("write your pallas kernel here")


# speedup vs baseline: 10.2639x; 10.2639x over previous
import math

import jax
import jax.numpy as jnp
from jax.experimental import pallas as pl
from jax.experimental.pallas import tpu as pltpu

_SIZE = 32000
_PAD = 0
_SMOOTH = 0.1
_CONF = 1.0 - _SMOOTH
_S = _SMOOTH / _SIZE
_KCONST = (_SIZE - 2) * _S * math.log(_S) + (_CONF + _S) * math.log(_CONF + _S)

_BR = 128


def _body(t_ref, x_ref, out_ref):
    i = pl.program_id(0)
    xb = x_ref[...]
    t = t_ref[0, pl.ds(i * _BR, _BR)]

    rowsum = jnp.sum(xb, axis=1)
    x0 = xb[:, 0]
    col = jax.lax.broadcasted_iota(jnp.int32, (_BR, _SIZE), 1)
    xt = jnp.sum(jnp.where(col == t[:, None], xb, 0.0), axis=1)

    contrib = jnp.where(
        t != _PAD,
        _KCONST - _S * (rowsum - x0) - _CONF * xt,
        0.0,
    )

    @pl.when(i == 0)
    def _init():
        out_ref[...] = jnp.zeros_like(out_ref)

    out_ref[...] += contrib.reshape(1, _BR)


def kernel(x, target):
    n, v = x.shape
    grid = n // _BR
    total = pl.pallas_call(
        _body,
        grid=(grid,),
        in_specs=[
            pl.BlockSpec((1, n), lambda i: (0, 0)),
            pl.BlockSpec((_BR, v), lambda i: (i, 0)),
        ],
        out_specs=pl.BlockSpec((1, _BR), lambda i: (0, 0)),
        out_shape=jax.ShapeDtypeStruct((1, _BR), jnp.float32),
        compiler_params=pltpu.CompilerParams(
            dimension_semantics=("arbitrary",),
        ),
    )(target.reshape(1, n), x)
    return jnp.sum(total) / n
